# baseline (speedup 1.0000x reference)
import jax
import jax.numpy as jnp
from jax.experimental import pallas as pl
from jax.experimental.pallas import tpu as pltpu


def kernel(x, pos_embed_weight):
    batch, seq, d = x.shape
    s_blk = min(1024, seq)
    n_seq = seq // s_blk
    grid = (n_seq, batch)
    w = pos_embed_weight[:seq]

    def body(w_hbm, x_ref, o_ref, w_vmem, sem):
        i = pl.program_id(0)
        j = pl.program_id(1)

        def copy_slice(k):
            src = w_hbm.at[pl.ds(k * s_blk, s_blk), :]
            dst = w_vmem.at[pl.ds(k * s_blk, s_blk), :]
            return pltpu.make_async_copy(src, dst, sem)

        @pl.when((i == 0) & (j == 0))
        def _():
            cp = copy_slice(0)
            cp.start()
            cp.wait()
            if n_seq > 1:
                copy_slice(1).start()

        @pl.when((i > 0) & (j == 0))
        def _():
            copy_slice(i).wait()

            @pl.when(i < n_seq - 1)
            def _():
                copy_slice(i + 1).start()

        o_ref[...] = x_ref[...] + w_vmem[pl.ds(i * s_blk, s_blk), :][None, :, :]

    out = pl.pallas_call(
        body,
        grid=grid,
        in_specs=[
            pl.BlockSpec(memory_space=pl.MemorySpace.ANY),
            pl.BlockSpec((1, s_blk, d), lambda i, j: (j, i, 0)),
        ],
        out_specs=pl.BlockSpec((1, s_blk, d), lambda i, j: (j, i, 0)),
        out_shape=jax.ShapeDtypeStruct((batch, seq, d), x.dtype),
        scratch_shapes=[
            pltpu.VMEM((seq, d), x.dtype),
            pltpu.SemaphoreType.DMA,
        ],
    )(w, x)
    return out


# speedup vs baseline: 1.0347x; 1.0347x over previous
import jax
import jax.numpy as jnp
from jax.experimental import pallas as pl
from jax.experimental.pallas import tpu as pltpu


def _add_body(x_ref, w_ref, o_ref):
    o_ref[...] = x_ref[...] + w_ref[...][None, :, :]


def kernel(x, pos_embed_weight):
    batch, seq, d = x.shape
    s_blk = min(512, seq)
    n_seq = seq // s_blk
    out = pl.pallas_call(
        _add_body,
        grid=(n_seq,),
        in_specs=[
            pl.BlockSpec((batch, s_blk, d), lambda i: (0, i, 0)),
            pl.BlockSpec((s_blk, d), lambda i: (i, 0)),
        ],
        out_specs=pl.BlockSpec((batch, s_blk, d), lambda i: (0, i, 0)),
        out_shape=jax.ShapeDtypeStruct((batch, seq, d), x.dtype),
    )(x, pos_embed_weight[:seq])
    return out
